# baseline (speedup 1.0000x reference)
import jax
import jax.numpy as jnp
from jax import lax
from jax.experimental import pallas as pl
from jax.experimental.pallas import tpu as pltpu
from jax.experimental.pallas import tpu_sc as plsc

N = 10000
E = 320000
D = 128
NLAYERS = 3

NC = 2
NS = 16
NW = NC * NS
CH = 40
EPW = E // NW
NCHUNK = EPW // CH
DW = D + 16
NPAD = 10240
RPS = NPAD // NS
_G_OFFS = (0, 16, 24)

_MESH = plsc.VectorSubcoreMesh(core_axis_name="c", subcore_axis_name="s")
_SC_PARAMS = pltpu.CompilerParams(use_tc_tiling_on_sc=False)


def _zero_fill_2d(ref, nrows, ncols):
    zero16 = jnp.zeros((16,), jnp.float32)

    def body(i, carry):
        for cg in range(ncols // 16):
            ref[i, pl.ds(cg * 16, 16)] = zero16
        return carry

    lax.fori_loop(0, nrows, body, 0)


def _unpack_idx(pk_all, i, isrc, idst):
    for off in _G_OFFS:
        v = pk_all[i, pl.ds(off, 16)]
        isrc[pl.ds(off, 16)] = v & jnp.int32(0xFFFF)
        idst[pl.ds(off, 16)] = lax.shift_right_logical(v, 16)


def _sc_deg_body(pkw, outdeg_hbm, indeg_hbm,
                 pk_all, is0, is1, is2, is3, id0, id1, id2, id3,
                 ones_v, zbuf, ssem0, ssem1, ssem2, ssem3, od_sh, id_sh):
    c = lax.axis_index("c")
    s = lax.axis_index("s")
    wid = c * NS + s

    pltpu.sync_copy(pkw.at[wid], pk_all)

    one16 = jnp.ones((16,), jnp.float32)

    def fill_ones(i, carry):
        ones_v[i, pl.ds(0, 16)] = one16
        return carry
    lax.fori_loop(0, CH, fill_ones, 0)

    _zero_fill_2d(zbuf, CH, 16)

    def zinit(k, carry):
        r0 = s * RPS + k * CH
        pltpu.async_copy(zbuf, od_sh.at[pl.ds(r0, CH)], ssem0)
        pltpu.async_copy(zbuf, id_sh.at[pl.ds(r0, CH)], ssem0)
        return carry
    lax.fori_loop(0, RPS // CH, zinit, 0)

    def zdrain(k, carry):
        r0 = s * RPS + k * CH
        pltpu.make_async_copy(zbuf, od_sh.at[pl.ds(r0, CH)], ssem0).wait()
        pltpu.make_async_copy(zbuf, id_sh.at[pl.ds(r0, CH)], ssem0).wait()
        return carry
    lax.fori_loop(0, RPS // CH, zdrain, 0)
    plsc.subcore_barrier()

    isrc = (is0, is1, is2, is3)
    idst = (id0, id1, id2, id3)
    ssem = (ssem0, ssem1, ssem2, ssem3)

    def wait_pair(b):
        pltpu.make_async_copy(ones_v, od_sh.at[isrc[b]], ssem[b]).wait()
        pltpu.make_async_copy(ones_v, id_sh.at[idst[b]], ssem[b]).wait()

    def quad(q, carry):
        for b in range(4):
            i = q * 4 + b

            @pl.when(i < NCHUNK)
            def _():
                @pl.when(i >= 3)
                def _():
                    wait_pair((b + 1) % 4)
                _unpack_idx(pk_all, i, isrc[b], idst[b])
                pltpu.async_copy(ones_v, od_sh.at[isrc[b]], ssem[b], add=True)
                pltpu.async_copy(ones_v, id_sh.at[idst[b]], ssem[b], add=True)
        return carry
    lax.fori_loop(0, (NCHUNK + 3) // 4, quad, 0)
    for j in range(NCHUNK - 3, NCHUNK):
        wait_pair(j % 4)

    plsc.subcore_barrier()
    r0 = s * RPS
    pltpu.sync_copy(od_sh.at[pl.ds(r0, RPS)], outdeg_hbm.at[c, pl.ds(r0, RPS)])
    pltpu.sync_copy(id_sh.at[pl.ds(r0, RPS)], indeg_hbm.at[c, pl.ds(r0, RPS)])


_sc_deg = pl.kernel(
    _sc_deg_body,
    out_type=(
        jax.ShapeDtypeStruct((NC, NPAD, 16), jnp.float32),
        jax.ShapeDtypeStruct((NC, NPAD, 16), jnp.float32),
    ),
    mesh=_MESH,
    compiler_params=_SC_PARAMS,
    scratch_types=[
        pltpu.VMEM((NCHUNK, CH), jnp.int32),
        pltpu.VMEM((CH,), jnp.int32),
        pltpu.VMEM((CH,), jnp.int32),
        pltpu.VMEM((CH,), jnp.int32),
        pltpu.VMEM((CH,), jnp.int32),
        pltpu.VMEM((CH,), jnp.int32),
        pltpu.VMEM((CH,), jnp.int32),
        pltpu.VMEM((CH,), jnp.int32),
        pltpu.VMEM((CH,), jnp.int32),
        pltpu.VMEM((CH, 16), jnp.float32),
        pltpu.VMEM((CH, 16), jnp.float32),
        pltpu.SemaphoreType.DMA,
        pltpu.SemaphoreType.DMA,
        pltpu.SemaphoreType.DMA,
        pltpu.SemaphoreType.DMA,
        pltpu.VMEM_SHARED((NPAD, 16), jnp.float32),
        pltpu.VMEM_SHARED((NPAD, 16), jnp.float32),
    ],
)


def _sc_msg_body(pkw, h_hbm, out_hbm,
                 pk_all, is0, is1, is2, is3, id0, id1, id2, id3,
                 rows0, rows1, rows2, rows3,
                 gsem0, gsem1, gsem2, gsem3, ssem0, ssem1, ssem2, ssem3,
                 acc_sh):
    c = lax.axis_index("c")
    s = lax.axis_index("s")
    wid = c * NS + s

    pltpu.sync_copy(pkw.at[wid], pk_all)

    _zero_fill_2d(rows0, CH, D)

    def zinit(k, carry):
        r0 = s * RPS + k * CH
        pltpu.async_copy(rows0, acc_sh.at[pl.ds(r0, CH)], gsem0)
        return carry
    lax.fori_loop(0, RPS // CH, zinit, 0)

    def zdrain(k, carry):
        r0 = s * RPS + k * CH
        pltpu.make_async_copy(rows0, acc_sh.at[pl.ds(r0, CH)], gsem0).wait()
        return carry
    lax.fori_loop(0, RPS // CH, zdrain, 0)
    plsc.subcore_barrier()

    isrc = (is0, is1, is2, is3)
    idst = (id0, id1, id2, id3)
    rows = (rows0, rows1, rows2, rows3)
    gsem = (gsem0, gsem1, gsem2, gsem3)
    ssem = (ssem0, ssem1, ssem2, ssem3)

    def issue_gather(i, b):
        _unpack_idx(pk_all, i, isrc[b], idst[b])
        pltpu.async_copy(h_hbm.at[isrc[b]], rows[b], gsem[b])

    def wait_gather(b):
        pltpu.make_async_copy(h_hbm.at[isrc[b]], rows[b], gsem[b]).wait()

    def issue_scatter(b):
        pltpu.async_copy(rows[b], acc_sh.at[idst[b]], ssem[b], add=True)

    def wait_scatter(b):
        pltpu.make_async_copy(rows[b], acc_sh.at[idst[b]], ssem[b]).wait()

    issue_gather(0, 0)
    issue_gather(1, 1)

    LASTC = NCHUNK - 1

    def quad(q, carry):
        for b in range(4):
            i = q * 4 + b

            @pl.when(i <= LASTC)
            def _():
                @pl.when(i >= 2)
                def _():
                    wait_scatter((b + 2) % 4)

                @pl.when(i + 2 <= LASTC)
                def _():
                    issue_gather(i + 2, (b + 2) % 4)
                wait_gather(b)
                issue_scatter(b)
        return carry
    lax.fori_loop(0, (NCHUNK + 3) // 4, quad, 0)
    wait_scatter((NCHUNK - 2) % 4)
    wait_scatter((NCHUNK - 1) % 4)

    plsc.subcore_barrier()
    r0 = s * RPS
    pltpu.sync_copy(acc_sh.at[pl.ds(r0, RPS)], out_hbm.at[c, pl.ds(r0, RPS)])


_sc_msg = pl.kernel(
    _sc_msg_body,
    out_type=jax.ShapeDtypeStruct((NC, NPAD, D), jnp.float32),
    mesh=_MESH,
    compiler_params=_SC_PARAMS,
    scratch_types=[
        pltpu.VMEM((NCHUNK, CH), jnp.int32),
        pltpu.VMEM((CH,), jnp.int32),
        pltpu.VMEM((CH,), jnp.int32),
        pltpu.VMEM((CH,), jnp.int32),
        pltpu.VMEM((CH,), jnp.int32),
        pltpu.VMEM((CH,), jnp.int32),
        pltpu.VMEM((CH,), jnp.int32),
        pltpu.VMEM((CH,), jnp.int32),
        pltpu.VMEM((CH,), jnp.int32),
        pltpu.VMEM((CH, D), jnp.float32),
        pltpu.VMEM((CH, D), jnp.float32),
        pltpu.VMEM((CH, D), jnp.float32),
        pltpu.VMEM((CH, D), jnp.float32),
        pltpu.SemaphoreType.DMA,
        pltpu.SemaphoreType.DMA,
        pltpu.SemaphoreType.DMA,
        pltpu.SemaphoreType.DMA,
        pltpu.SemaphoreType.DMA,
        pltpu.SemaphoreType.DMA,
        pltpu.SemaphoreType.DMA,
        pltpu.SemaphoreType.DMA,
        pltpu.VMEM_SHARED((NPAD, D), jnp.float32),
    ],
)


def _sc_gat_body(pkw, zel_hbm, b32_hbm, acc_hbm,
                 pk_all, is0, is1, is2, is3, id0, id1, id2, id3,
                 rows0, rows1, rows2, rows3, bb0, bb1, bb2, bb3,
                 gsem0, gsem1, gsem2, gsem3, ssem0, ssem1, ssem2, ssem3,
                 acc_sh):
    c = lax.axis_index("c")
    s = lax.axis_index("s")
    wid = c * NS + s

    pltpu.sync_copy(pkw.at[wid], pk_all)

    _zero_fill_2d(rows0, CH, DW)

    def zinit(k, carry):
        r0 = s * RPS + k * CH
        pltpu.async_copy(rows0, acc_sh.at[pl.ds(r0, CH)], gsem0)
        return carry
    lax.fori_loop(0, RPS // CH, zinit, 0)

    def zdrain(k, carry):
        r0 = s * RPS + k * CH
        pltpu.make_async_copy(rows0, acc_sh.at[pl.ds(r0, CH)], gsem0).wait()
        return carry
    lax.fori_loop(0, RPS // CH, zdrain, 0)
    plsc.subcore_barrier()

    isrc = (is0, is1, is2, is3)
    idst = (id0, id1, id2, id3)
    rows = (rows0, rows1, rows2, rows3)
    bb = (bb0, bb1, bb2, bb3)
    gsem = (gsem0, gsem1, gsem2, gsem3)
    ssem = (ssem0, ssem1, ssem2, ssem3)

    def issue_gather(i, b):
        _unpack_idx(pk_all, i, isrc[b], idst[b])
        pltpu.async_copy(zel_hbm.at[isrc[b]], rows[b], gsem[b])
        pltpu.async_copy(b32_hbm.at[idst[b]], bb[b], gsem[b])

    def wait_gather(b):
        pltpu.make_async_copy(zel_hbm.at[isrc[b]], rows[b], gsem[b]).wait()
        pltpu.make_async_copy(b32_hbm.at[idst[b]], bb[b], gsem[b]).wait()

    def issue_scatter(b):
        pltpu.async_copy(rows[b], acc_sh.at[idst[b]], ssem[b], add=True)

    def wait_scatter(b):
        pltpu.make_async_copy(rows[b], acc_sh.at[idst[b]], ssem[b]).wait()

    def scale(b):
        @plsc.parallel_loop(0, CH, 1, unroll=16)
        def _(e):
            elr16 = rows[b][e, pl.ds(D, 16)]
            err16 = bb[b][e, pl.ds(0, 16)]
            tr16 = bb[b][e, pl.ds(16, 16)]
            x = elr16 + err16
            ee = jnp.where(x >= 0.0, x, 0.2 * x)
            w = jnp.exp(ee + tr16)
            rows[b][e, pl.ds(D, 16)] = w
            for cg in range(D // 16):
                rows[b][e, pl.ds(cg * 16, 16)] = rows[b][e, pl.ds(cg * 16, 16)] * w

    issue_gather(0, 0)
    issue_gather(1, 1)

    LASTC = NCHUNK - 1

    def quad(q, carry):
        for b in range(4):
            i = q * 4 + b

            @pl.when(i <= LASTC)
            def _():
                @pl.when(i >= 2)
                def _():
                    wait_scatter((b + 2) % 4)

                @pl.when(i + 2 <= LASTC)
                def _():
                    issue_gather(i + 2, (b + 2) % 4)
                wait_gather(b)
                scale(b)
                issue_scatter(b)
        return carry
    lax.fori_loop(0, (NCHUNK + 3) // 4, quad, 0)
    wait_scatter((NCHUNK - 2) % 4)
    wait_scatter((NCHUNK - 1) % 4)

    plsc.subcore_barrier()
    r0 = s * RPS
    pltpu.sync_copy(acc_sh.at[pl.ds(r0, RPS)], acc_hbm.at[c, pl.ds(r0, RPS)])


_sc_gat = pl.kernel(
    _sc_gat_body,
    out_type=jax.ShapeDtypeStruct((NC, NPAD, DW), jnp.float32),
    mesh=_MESH,
    compiler_params=_SC_PARAMS,
    scratch_types=[
        pltpu.VMEM((NCHUNK, CH), jnp.int32),
        pltpu.VMEM((CH,), jnp.int32),
        pltpu.VMEM((CH,), jnp.int32),
        pltpu.VMEM((CH,), jnp.int32),
        pltpu.VMEM((CH,), jnp.int32),
        pltpu.VMEM((CH,), jnp.int32),
        pltpu.VMEM((CH,), jnp.int32),
        pltpu.VMEM((CH,), jnp.int32),
        pltpu.VMEM((CH,), jnp.int32),
        pltpu.VMEM((CH, DW), jnp.float32),
        pltpu.VMEM((CH, DW), jnp.float32),
        pltpu.VMEM((CH, DW), jnp.float32),
        pltpu.VMEM((CH, DW), jnp.float32),
        pltpu.VMEM((CH, 32), jnp.float32),
        pltpu.VMEM((CH, 32), jnp.float32),
        pltpu.VMEM((CH, 32), jnp.float32),
        pltpu.VMEM((CH, 32), jnp.float32),
        pltpu.SemaphoreType.DMA,
        pltpu.SemaphoreType.DMA,
        pltpu.SemaphoreType.DMA,
        pltpu.SemaphoreType.DMA,
        pltpu.SemaphoreType.DMA,
        pltpu.SemaphoreType.DMA,
        pltpu.SemaphoreType.DMA,
        pltpu.SemaphoreType.DMA,
        pltpu.VMEM_SHARED((NPAD, DW), jnp.float32),
    ],
)


def _tc_prescale_body(x_ref, od_ref, id_ref, sx_ref, rsi_ref, rso_ref):
    outd = od_ref[0, :N, 0:1] + od_ref[1, :N, 0:1] + 1.0
    ind = id_ref[0, :N, 0:1] + id_ref[1, :N, 0:1] + 1.0
    rso = lax.rsqrt(jnp.maximum(outd, 1.0))
    rsi = lax.rsqrt(jnp.maximum(ind, 1.0))
    rso_ref[...] = rso
    rsi_ref[...] = rsi
    sx_ref[...] = x_ref[...] * rso


_tc_prescale = pl.pallas_call(
    _tc_prescale_body,
    out_shape=(
        jax.ShapeDtypeStruct((N, D), jnp.float32),
        jax.ShapeDtypeStruct((N, 1), jnp.float32),
        jax.ShapeDtypeStruct((N, 1), jnp.float32),
    ),
)


def _tc_gcn_post_body(p_ref, sx_ref, rsi_ref, w_ref, b_ref, h_ref, r_ref):
    m = (p_ref[0, :N] + p_ref[1, :N] + sx_ref[...]) * rsi_ref[...]
    h = jnp.maximum(jnp.dot(m, w_ref[...], preferred_element_type=jnp.float32)
                    + b_ref[...], 0.0)
    h_ref[...] = h
    r_ref[...] = jnp.concatenate(
        [jnp.mean(h, axis=0)[None, :], jnp.max(h, axis=0)[None, :]], axis=1)


_tc_gcn_post = pl.pallas_call(
    _tc_gcn_post_body,
    out_shape=(
        jax.ShapeDtypeStruct((N, D), jnp.float32),
        jax.ShapeDtypeStruct((1, 2 * D), jnp.float32),
    ),
)


def _tc_gat_pre_body(h_ref, r_ref, supw_ref, supb_ref, gatw_ref, al_ref, ar_ref,
                     zel_ref, b32_ref, wsup_ref, zs_ref):
    sfeat = jnp.maximum(
        jnp.dot(r_ref[...], supw_ref[...], preferred_element_type=jnp.float32)
        + supb_ref[...], 0.0)
    z = jnp.dot(h_ref[...], gatw_ref[...], preferred_element_type=jnp.float32)
    zs = jnp.dot(sfeat, gatw_ref[...], preferred_element_type=jnp.float32)
    el = jnp.dot(z, al_ref[...], preferred_element_type=jnp.float32)
    er = jnp.dot(z, ar_ref[...], preferred_element_type=jnp.float32)
    els = jnp.dot(zs, al_ref[...], preferred_element_type=jnp.float32)[0, 0]
    big_m = jnp.maximum(jnp.max(el), els)
    xm = big_m + er
    c = jnp.where(xm >= 0.0, xm, 0.2 * xm)
    xs = els + er
    esup = jnp.where(xs >= 0.0, xs, 0.2 * xs)
    ones16 = jnp.ones((1, 16), jnp.float32)
    zel_ref[...] = jnp.concatenate([z, el * ones16], axis=1)
    b32_ref[...] = jnp.concatenate([er * ones16, (-c) * ones16], axis=1)
    wsup_ref[...] = jnp.exp(esup - c)
    zs_ref[...] = zs


_tc_gat_pre = pl.pallas_call(
    _tc_gat_pre_body,
    out_shape=(
        jax.ShapeDtypeStruct((N, DW), jnp.float32),
        jax.ShapeDtypeStruct((N, 32), jnp.float32),
        jax.ShapeDtypeStruct((N, 1), jnp.float32),
        jax.ShapeDtypeStruct((1, D), jnp.float32),
    ),
)


def _tc_gat_post_body(acc_ref, wsup_ref, zs_ref, rso_ref, sx_ref):
    wsup = wsup_ref[...]
    num = acc_ref[0, :N, 0:D] + acc_ref[1, :N, 0:D] + wsup * zs_ref[...]
    den = acc_ref[0, :N, D:D + 1] + acc_ref[1, :N, D:D + 1] + wsup
    h = num / jnp.maximum(den, 1e-30)
    sx_ref[...] = h * rso_ref[...]


_tc_gat_post = pl.pallas_call(
    _tc_gat_post_body,
    out_shape=jax.ShapeDtypeStruct((N, D), jnp.float32),
)


def _tc_final_body(r0_ref, r1_ref, r2_ref, w1_ref, b1_ref, w2_ref, b2_ref,
                   w3_ref, b3_ref, out_ref):
    n_feat = jnp.concatenate([r0_ref[...], r1_ref[...], r2_ref[...]], axis=1)
    h1 = jnp.maximum(
        jnp.dot(n_feat, w1_ref[...], preferred_element_type=jnp.float32)
        + b1_ref[...], 0.0)
    h2 = jnp.maximum(
        jnp.dot(h1, w2_ref[...], preferred_element_type=jnp.float32)
        + b2_ref[...], 0.0)
    h3 = jnp.dot(h2, w3_ref[...], preferred_element_type=jnp.float32) + b3_ref[...]
    m = jnp.max(h3, axis=1, keepdims=True)
    lse = m + jnp.log(jnp.sum(jnp.exp(h3 - m), axis=1, keepdims=True))
    out_ref[...] = h3 - lse


_tc_final = pl.pallas_call(
    _tc_final_body,
    out_shape=jax.ShapeDtypeStruct((1, 2), jnp.float32),
)


def kernel(x0, x1, x2, edge_index0, edge_index1, edge_index2, params):
    p = params
    xs = [x0, x1, x2]
    pks = []
    for e in [edge_index0, edge_index1, edge_index2]:
        s32 = e[0].astype(jnp.int32)
        d32 = e[1].astype(jnp.int32)
        pks.append(((d32 << 16) | s32).reshape(NW, NCHUNK, CH))

    sx = [None] * 3
    rsi = [None] * 3
    rso = [None] * 3
    for g in range(3):
        od_p, id_p = _sc_deg(pks[g])
        sx[g], rsi[g], rso[g] = _tc_prescale(xs[g], od_p, id_p)

    readouts = [None] * 3
    hs = [None] * 3
    for i in range(NLAYERS - 1):
        for g in range(3):
            m_p = _sc_msg(pks[g], sx[g])
            hs[g], readouts[g] = _tc_gcn_post(
                m_p, sx[g], rsi[g],
                p['convW_%d_%d' % (g, i)],
                p['convb_%d_%d' % (g, i)].reshape(1, D))
        if i % 2 == 0:
            wiring = [(1, 'g2s'), (2, 't2g'), (0, 's2t')]
        else:
            wiring = [(2, 't2s'), (0, 's2g'), (1, 'g2t')]
        for g in range(3):
            r_src, wname = wiring[g]
            zel, b32, wsup, zs = _tc_gat_pre(
                hs[g], readouts[r_src],
                p[wname + '_W'], p[wname + '_b'].reshape(1, D),
                p['gatW_%d' % g],
                p['gat_al_%d' % g].reshape(D, 1),
                p['gat_ar_%d' % g].reshape(D, 1))
            acc_p = _sc_gat(pks[g], zel, b32)
            sx[g] = _tc_gat_post(acc_p, wsup, zs, rso[g])

    last = NLAYERS - 1
    for g in range(3):
        m_p = _sc_msg(pks[g], sx[g])
        _, readouts[g] = _tc_gcn_post(
            m_p, sx[g], rsi[g],
            p['convW_%d_%d' % (g, last)],
            p['convb_%d_%d' % (g, last)].reshape(1, D))

    return _tc_final(
        readouts[0], readouts[1], readouts[2],
        p['lin1_W'], p['lin1_b'].reshape(1, -1),
        p['lin2_W'], p['lin2_b'].reshape(1, -1),
        p['lin3_W'], p['lin3_b'].reshape(1, -1))


# speedup vs baseline: 1.0803x; 1.0803x over previous
import jax
import jax.numpy as jnp
from jax import lax
from jax.experimental import pallas as pl
from jax.experimental.pallas import tpu as pltpu
from jax.experimental.pallas import tpu_sc as plsc

N = 10000
E = 320000
D = 128
NLAYERS = 3

NC = 2
NS = 16
NW = NC * NS
CH = 40
EPW = E // NW
NCHUNK = EPW // CH
DW = D + 16
NPAD = 10240
RPS = NPAD // NS
_G_OFFS = (0, 16, 24)

_MESH = plsc.VectorSubcoreMesh(core_axis_name="c", subcore_axis_name="s")
_SC_PARAMS = pltpu.CompilerParams(use_tc_tiling_on_sc=False)


def _zero_fill_2d(ref, nrows, ncols):
    zero16 = jnp.zeros((16,), jnp.float32)

    def body(i, carry):
        for cg in range(ncols // 16):
            ref[i, pl.ds(cg * 16, 16)] = zero16
        return carry

    lax.fori_loop(0, nrows, body, 0)


def _unpack_idx(pk_all, i, isrc, idst):
    for off in _G_OFFS:
        v = pk_all[i, pl.ds(off, 16)]
        isrc[pl.ds(off, 16)] = v & jnp.int32(0xFFFF)
        idst[pl.ds(off, 16)] = lax.shift_right_logical(v, 16)


def _sc_deg_body(pkw, outdeg_hbm, indeg_hbm,
                 pk_all, is0, is1, is2, is3, id0, id1, id2, id3,
                 ones_v, zbuf, ssem0, ssem1, ssem2, ssem3, od_sh, id_sh):
    c = lax.axis_index("c")
    s = lax.axis_index("s")
    wid = c * NS + s

    pltpu.sync_copy(pkw.at[wid], pk_all)

    one16 = jnp.ones((16,), jnp.float32)

    def fill_ones(i, carry):
        ones_v[i, pl.ds(0, 16)] = one16
        return carry
    lax.fori_loop(0, CH, fill_ones, 0)

    _zero_fill_2d(zbuf, CH, 16)

    def zinit(k, carry):
        r0 = s * RPS + k * CH
        pltpu.async_copy(zbuf, od_sh.at[pl.ds(r0, CH)], ssem0)
        pltpu.async_copy(zbuf, id_sh.at[pl.ds(r0, CH)], ssem0)
        return carry
    lax.fori_loop(0, RPS // CH, zinit, 0)

    def zdrain(k, carry):
        r0 = s * RPS + k * CH
        pltpu.make_async_copy(zbuf, od_sh.at[pl.ds(r0, CH)], ssem0).wait()
        pltpu.make_async_copy(zbuf, id_sh.at[pl.ds(r0, CH)], ssem0).wait()
        return carry
    lax.fori_loop(0, RPS // CH, zdrain, 0)
    plsc.subcore_barrier()

    isrc = (is0, is1, is2, is3)
    idst = (id0, id1, id2, id3)
    ssem = (ssem0, ssem1, ssem2, ssem3)

    def wait_pair(b):
        pltpu.make_async_copy(ones_v, od_sh.at[isrc[b]], ssem[b]).wait()
        pltpu.make_async_copy(ones_v, id_sh.at[idst[b]], ssem[b]).wait()

    def quad(q, carry):
        for b in range(4):
            i = q * 4 + b

            @pl.when(i < NCHUNK)
            def _():
                @pl.when(i >= 3)
                def _():
                    wait_pair((b + 1) % 4)
                _unpack_idx(pk_all, i, isrc[b], idst[b])
                pltpu.async_copy(ones_v, od_sh.at[isrc[b]], ssem[b], add=True)
                pltpu.async_copy(ones_v, id_sh.at[idst[b]], ssem[b], add=True)
        return carry
    lax.fori_loop(0, (NCHUNK + 3) // 4, quad, 0)
    for j in range(NCHUNK - 3, NCHUNK):
        wait_pair(j % 4)

    plsc.subcore_barrier()
    r0 = s * RPS
    pltpu.sync_copy(od_sh.at[pl.ds(r0, RPS)], outdeg_hbm.at[c, pl.ds(r0, RPS)])
    pltpu.sync_copy(id_sh.at[pl.ds(r0, RPS)], indeg_hbm.at[c, pl.ds(r0, RPS)])


_sc_deg = pl.kernel(
    _sc_deg_body,
    out_type=(
        jax.ShapeDtypeStruct((NC, NPAD, 16), jnp.float32),
        jax.ShapeDtypeStruct((NC, NPAD, 16), jnp.float32),
    ),
    mesh=_MESH,
    compiler_params=_SC_PARAMS,
    scratch_types=[
        pltpu.VMEM((NCHUNK, CH), jnp.int32),
        pltpu.VMEM((CH,), jnp.int32),
        pltpu.VMEM((CH,), jnp.int32),
        pltpu.VMEM((CH,), jnp.int32),
        pltpu.VMEM((CH,), jnp.int32),
        pltpu.VMEM((CH,), jnp.int32),
        pltpu.VMEM((CH,), jnp.int32),
        pltpu.VMEM((CH,), jnp.int32),
        pltpu.VMEM((CH,), jnp.int32),
        pltpu.VMEM((CH, 16), jnp.float32),
        pltpu.VMEM((CH, 16), jnp.float32),
        pltpu.SemaphoreType.DMA,
        pltpu.SemaphoreType.DMA,
        pltpu.SemaphoreType.DMA,
        pltpu.SemaphoreType.DMA,
        pltpu.VMEM_SHARED((NPAD, 16), jnp.float32),
        pltpu.VMEM_SHARED((NPAD, 16), jnp.float32),
    ],
)


def _sc_msg_body(pkw, h_hbm, out_hbm,
                 pk_all, is0, is1, is2, is3, id0, id1, id2, id3,
                 rows0, rows1, rows2, rows3,
                 gsem0, gsem1, gsem2, gsem3, ssem0, ssem1, ssem2, ssem3,
                 acc_sh):
    c = lax.axis_index("c")
    s = lax.axis_index("s")
    wid = c * NS + s

    pltpu.sync_copy(pkw.at[wid], pk_all)

    _zero_fill_2d(rows0, CH, D)

    def zinit(k, carry):
        r0 = s * RPS + k * CH
        pltpu.async_copy(rows0, acc_sh.at[pl.ds(r0, CH)], gsem0)
        return carry
    lax.fori_loop(0, RPS // CH, zinit, 0)

    def zdrain(k, carry):
        r0 = s * RPS + k * CH
        pltpu.make_async_copy(rows0, acc_sh.at[pl.ds(r0, CH)], gsem0).wait()
        return carry
    lax.fori_loop(0, RPS // CH, zdrain, 0)
    plsc.subcore_barrier()

    isrc = (is0, is1, is2, is3)
    idst = (id0, id1, id2, id3)
    rows = (rows0, rows1, rows2, rows3)
    gsem = (gsem0, gsem1, gsem2, gsem3)
    ssem = (ssem0, ssem1, ssem2, ssem3)

    def issue_gather(i, b):
        _unpack_idx(pk_all, i, isrc[b], idst[b])
        pltpu.async_copy(h_hbm.at[isrc[b]], rows[b], gsem[b])

    def wait_gather(b):
        pltpu.make_async_copy(h_hbm.at[isrc[b]], rows[b], gsem[b]).wait()

    def issue_scatter(b):
        pltpu.async_copy(rows[b], acc_sh.at[idst[b]], ssem[b], add=True)

    def wait_scatter(b):
        pltpu.make_async_copy(rows[b], acc_sh.at[idst[b]], ssem[b]).wait()

    issue_gather(0, 0)
    issue_gather(1, 1)

    LASTC = NCHUNK - 1

    def quad(q, carry):
        for b in range(4):
            i = q * 4 + b

            @pl.when(i <= LASTC)
            def _():
                @pl.when(i >= 2)
                def _():
                    wait_scatter((b + 2) % 4)

                @pl.when(i + 2 <= LASTC)
                def _():
                    issue_gather(i + 2, (b + 2) % 4)
                wait_gather(b)
                issue_scatter(b)
        return carry
    lax.fori_loop(0, (NCHUNK + 3) // 4, quad, 0)
    wait_scatter((NCHUNK - 2) % 4)
    wait_scatter((NCHUNK - 1) % 4)

    plsc.subcore_barrier()
    r0 = s * RPS
    pltpu.sync_copy(acc_sh.at[pl.ds(r0, RPS)], out_hbm.at[c, pl.ds(r0, RPS)])


_sc_msg = pl.kernel(
    _sc_msg_body,
    out_type=jax.ShapeDtypeStruct((NC, NPAD, D), jnp.float32),
    mesh=_MESH,
    compiler_params=_SC_PARAMS,
    scratch_types=[
        pltpu.VMEM((NCHUNK, CH), jnp.int32),
        pltpu.VMEM((CH,), jnp.int32),
        pltpu.VMEM((CH,), jnp.int32),
        pltpu.VMEM((CH,), jnp.int32),
        pltpu.VMEM((CH,), jnp.int32),
        pltpu.VMEM((CH,), jnp.int32),
        pltpu.VMEM((CH,), jnp.int32),
        pltpu.VMEM((CH,), jnp.int32),
        pltpu.VMEM((CH,), jnp.int32),
        pltpu.VMEM((CH, D), jnp.float32),
        pltpu.VMEM((CH, D), jnp.float32),
        pltpu.VMEM((CH, D), jnp.float32),
        pltpu.VMEM((CH, D), jnp.float32),
        pltpu.SemaphoreType.DMA,
        pltpu.SemaphoreType.DMA,
        pltpu.SemaphoreType.DMA,
        pltpu.SemaphoreType.DMA,
        pltpu.SemaphoreType.DMA,
        pltpu.SemaphoreType.DMA,
        pltpu.SemaphoreType.DMA,
        pltpu.SemaphoreType.DMA,
        pltpu.VMEM_SHARED((NPAD, D), jnp.float32),
    ],
)


def _sc_gat_body(pkw, zel_hbm, b32_hbm, acc_hbm,
                 pk_all, is0, is1, is2, is3, id0, id1, id2, id3,
                 rows0, rows1, rows2, rows3, bb0, bb1, bb2, bb3,
                 gsem0, gsem1, gsem2, gsem3, ssem0, ssem1, ssem2, ssem3,
                 acc_sh):
    c = lax.axis_index("c")
    s = lax.axis_index("s")
    wid = c * NS + s

    pltpu.sync_copy(pkw.at[wid], pk_all)

    _zero_fill_2d(rows0, CH, DW)

    def zinit(k, carry):
        r0 = s * RPS + k * CH
        pltpu.async_copy(rows0, acc_sh.at[pl.ds(r0, CH)], gsem0)
        return carry
    lax.fori_loop(0, RPS // CH, zinit, 0)

    def zdrain(k, carry):
        r0 = s * RPS + k * CH
        pltpu.make_async_copy(rows0, acc_sh.at[pl.ds(r0, CH)], gsem0).wait()
        return carry
    lax.fori_loop(0, RPS // CH, zdrain, 0)
    plsc.subcore_barrier()

    isrc = (is0, is1, is2, is3)
    idst = (id0, id1, id2, id3)
    rows = (rows0, rows1, rows2, rows3)
    bb = (bb0, bb1, bb2, bb3)
    gsem = (gsem0, gsem1, gsem2, gsem3)
    ssem = (ssem0, ssem1, ssem2, ssem3)

    def issue_gather(i, b):
        _unpack_idx(pk_all, i, isrc[b], idst[b])
        pltpu.async_copy(zel_hbm.at[isrc[b]], rows[b], gsem[b])
        pltpu.async_copy(b32_hbm.at[idst[b]], bb[b], gsem[b])

    def wait_gather(b):
        pltpu.make_async_copy(zel_hbm.at[isrc[b]], rows[b], gsem[b]).wait()
        pltpu.make_async_copy(b32_hbm.at[idst[b]], bb[b], gsem[b]).wait()

    def issue_scatter(b):
        pltpu.async_copy(rows[b], acc_sh.at[idst[b]], ssem[b], add=True)

    def wait_scatter(b):
        pltpu.make_async_copy(rows[b], acc_sh.at[idst[b]], ssem[b]).wait()

    def scale(b):
        @plsc.parallel_loop(0, CH, 1, unroll=4)
        def _(e):
            elr16 = rows[b][e, pl.ds(D, 16)]
            err16 = bb[b][e, pl.ds(0, 16)]
            tr16 = bb[b][e, pl.ds(16, 16)]
            x = elr16 + err16
            ee = jnp.where(x >= 0.0, x, 0.2 * x)
            w = jnp.exp(ee + tr16)
            rows[b][e, pl.ds(D, 16)] = w
            for cg in range(D // 16):
                rows[b][e, pl.ds(cg * 16, 16)] = rows[b][e, pl.ds(cg * 16, 16)] * w

    issue_gather(0, 0)
    issue_gather(1, 1)

    LASTC = NCHUNK - 1

    def quad(q, carry):
        for b in range(4):
            i = q * 4 + b

            @pl.when(i <= LASTC)
            def _():
                @pl.when(i >= 2)
                def _():
                    wait_scatter((b + 2) % 4)

                @pl.when(i + 2 <= LASTC)
                def _():
                    issue_gather(i + 2, (b + 2) % 4)
                wait_gather(b)
                scale(b)
                issue_scatter(b)
        return carry
    lax.fori_loop(0, (NCHUNK + 3) // 4, quad, 0)
    wait_scatter((NCHUNK - 2) % 4)
    wait_scatter((NCHUNK - 1) % 4)

    plsc.subcore_barrier()
    r0 = s * RPS
    pltpu.sync_copy(acc_sh.at[pl.ds(r0, RPS)], acc_hbm.at[c, pl.ds(r0, RPS)])


_sc_gat = pl.kernel(
    _sc_gat_body,
    out_type=jax.ShapeDtypeStruct((NC, NPAD, DW), jnp.float32),
    mesh=_MESH,
    compiler_params=_SC_PARAMS,
    scratch_types=[
        pltpu.VMEM((NCHUNK, CH), jnp.int32),
        pltpu.VMEM((CH,), jnp.int32),
        pltpu.VMEM((CH,), jnp.int32),
        pltpu.VMEM((CH,), jnp.int32),
        pltpu.VMEM((CH,), jnp.int32),
        pltpu.VMEM((CH,), jnp.int32),
        pltpu.VMEM((CH,), jnp.int32),
        pltpu.VMEM((CH,), jnp.int32),
        pltpu.VMEM((CH,), jnp.int32),
        pltpu.VMEM((CH, DW), jnp.float32),
        pltpu.VMEM((CH, DW), jnp.float32),
        pltpu.VMEM((CH, DW), jnp.float32),
        pltpu.VMEM((CH, DW), jnp.float32),
        pltpu.VMEM((CH, 32), jnp.float32),
        pltpu.VMEM((CH, 32), jnp.float32),
        pltpu.VMEM((CH, 32), jnp.float32),
        pltpu.VMEM((CH, 32), jnp.float32),
        pltpu.SemaphoreType.DMA,
        pltpu.SemaphoreType.DMA,
        pltpu.SemaphoreType.DMA,
        pltpu.SemaphoreType.DMA,
        pltpu.SemaphoreType.DMA,
        pltpu.SemaphoreType.DMA,
        pltpu.SemaphoreType.DMA,
        pltpu.SemaphoreType.DMA,
        pltpu.VMEM_SHARED((NPAD, DW), jnp.float32),
    ],
)


def _tc_prescale_body(x_ref, od_ref, id_ref, sx_ref, rsi_ref, rso_ref):
    outd = od_ref[0, :N, 0:1] + od_ref[1, :N, 0:1] + 1.0
    ind = id_ref[0, :N, 0:1] + id_ref[1, :N, 0:1] + 1.0
    rso = lax.rsqrt(jnp.maximum(outd, 1.0))
    rsi = lax.rsqrt(jnp.maximum(ind, 1.0))
    rso_ref[...] = rso
    rsi_ref[...] = rsi
    sx_ref[...] = x_ref[...] * rso


_tc_prescale = pl.pallas_call(
    _tc_prescale_body,
    out_shape=(
        jax.ShapeDtypeStruct((N, D), jnp.float32),
        jax.ShapeDtypeStruct((N, 1), jnp.float32),
        jax.ShapeDtypeStruct((N, 1), jnp.float32),
    ),
)


def _tc_gcn_post_body(p_ref, sx_ref, rsi_ref, w_ref, b_ref, h_ref, r_ref):
    m = (p_ref[0, :N] + p_ref[1, :N] + sx_ref[...]) * rsi_ref[...]
    h = jnp.maximum(jnp.dot(m, w_ref[...], preferred_element_type=jnp.float32)
                    + b_ref[...], 0.0)
    h_ref[...] = h
    r_ref[...] = jnp.concatenate(
        [jnp.mean(h, axis=0)[None, :], jnp.max(h, axis=0)[None, :]], axis=1)


_tc_gcn_post = pl.pallas_call(
    _tc_gcn_post_body,
    out_shape=(
        jax.ShapeDtypeStruct((N, D), jnp.float32),
        jax.ShapeDtypeStruct((1, 2 * D), jnp.float32),
    ),
)


def _tc_gat_pre_body(h_ref, r_ref, supw_ref, supb_ref, gatw_ref, al_ref, ar_ref,
                     zel_ref, b32_ref, wsup_ref, zs_ref):
    sfeat = jnp.maximum(
        jnp.dot(r_ref[...], supw_ref[...], preferred_element_type=jnp.float32)
        + supb_ref[...], 0.0)
    z = jnp.dot(h_ref[...], gatw_ref[...], preferred_element_type=jnp.float32)
    zs = jnp.dot(sfeat, gatw_ref[...], preferred_element_type=jnp.float32)
    el = jnp.dot(z, al_ref[...], preferred_element_type=jnp.float32)
    er = jnp.dot(z, ar_ref[...], preferred_element_type=jnp.float32)
    els = jnp.dot(zs, al_ref[...], preferred_element_type=jnp.float32)[0, 0]
    big_m = jnp.maximum(jnp.max(el), els)
    xm = big_m + er
    c = jnp.where(xm >= 0.0, xm, 0.2 * xm)
    xs = els + er
    esup = jnp.where(xs >= 0.0, xs, 0.2 * xs)
    ones16 = jnp.ones((1, 16), jnp.float32)
    zel_ref[...] = jnp.concatenate([z, el * ones16], axis=1)
    b32_ref[...] = jnp.concatenate([er * ones16, (-c) * ones16], axis=1)
    wsup_ref[...] = jnp.exp(esup - c)
    zs_ref[...] = zs


_tc_gat_pre = pl.pallas_call(
    _tc_gat_pre_body,
    out_shape=(
        jax.ShapeDtypeStruct((N, DW), jnp.float32),
        jax.ShapeDtypeStruct((N, 32), jnp.float32),
        jax.ShapeDtypeStruct((N, 1), jnp.float32),
        jax.ShapeDtypeStruct((1, D), jnp.float32),
    ),
)


def _tc_gat_post_body(acc_ref, wsup_ref, zs_ref, rso_ref, sx_ref):
    wsup = wsup_ref[...]
    num = acc_ref[0, :N, 0:D] + acc_ref[1, :N, 0:D] + wsup * zs_ref[...]
    den = acc_ref[0, :N, D:D + 1] + acc_ref[1, :N, D:D + 1] + wsup
    h = num / jnp.maximum(den, 1e-30)
    sx_ref[...] = h * rso_ref[...]


_tc_gat_post = pl.pallas_call(
    _tc_gat_post_body,
    out_shape=jax.ShapeDtypeStruct((N, D), jnp.float32),
)


def _tc_final_body(r0_ref, r1_ref, r2_ref, w1_ref, b1_ref, w2_ref, b2_ref,
                   w3_ref, b3_ref, out_ref):
    n_feat = jnp.concatenate([r0_ref[...], r1_ref[...], r2_ref[...]], axis=1)
    h1 = jnp.maximum(
        jnp.dot(n_feat, w1_ref[...], preferred_element_type=jnp.float32)
        + b1_ref[...], 0.0)
    h2 = jnp.maximum(
        jnp.dot(h1, w2_ref[...], preferred_element_type=jnp.float32)
        + b2_ref[...], 0.0)
    h3 = jnp.dot(h2, w3_ref[...], preferred_element_type=jnp.float32) + b3_ref[...]
    m = jnp.max(h3, axis=1, keepdims=True)
    lse = m + jnp.log(jnp.sum(jnp.exp(h3 - m), axis=1, keepdims=True))
    out_ref[...] = h3 - lse


_tc_final = pl.pallas_call(
    _tc_final_body,
    out_shape=jax.ShapeDtypeStruct((1, 2), jnp.float32),
)


def kernel(x0, x1, x2, edge_index0, edge_index1, edge_index2, params):
    p = params
    xs = [x0, x1, x2]
    pks = []
    for e in [edge_index0, edge_index1, edge_index2]:
        s32 = e[0].astype(jnp.int32)
        d32 = e[1].astype(jnp.int32)
        pks.append(((d32 << 16) | s32).reshape(NW, NCHUNK, CH))

    sx = [None] * 3
    rsi = [None] * 3
    rso = [None] * 3
    for g in range(3):
        od_p, id_p = _sc_deg(pks[g])
        sx[g], rsi[g], rso[g] = _tc_prescale(xs[g], od_p, id_p)

    readouts = [None] * 3
    hs = [None] * 3
    for i in range(NLAYERS - 1):
        for g in range(3):
            m_p = _sc_msg(pks[g], sx[g])
            hs[g], readouts[g] = _tc_gcn_post(
                m_p, sx[g], rsi[g],
                p['convW_%d_%d' % (g, i)],
                p['convb_%d_%d' % (g, i)].reshape(1, D))
        if i % 2 == 0:
            wiring = [(1, 'g2s'), (2, 't2g'), (0, 's2t')]
        else:
            wiring = [(2, 't2s'), (0, 's2g'), (1, 'g2t')]
        for g in range(3):
            r_src, wname = wiring[g]
            zel, b32, wsup, zs = _tc_gat_pre(
                hs[g], readouts[r_src],
                p[wname + '_W'], p[wname + '_b'].reshape(1, D),
                p['gatW_%d' % g],
                p['gat_al_%d' % g].reshape(D, 1),
                p['gat_ar_%d' % g].reshape(D, 1))
            acc_p = _sc_gat(pks[g], zel, b32)
            sx[g] = _tc_gat_post(acc_p, wsup, zs, rso[g])

    last = NLAYERS - 1
    for g in range(3):
        m_p = _sc_msg(pks[g], sx[g])
        _, readouts[g] = _tc_gcn_post(
            m_p, sx[g], rsi[g],
            p['convW_%d_%d' % (g, last)],
            p['convb_%d_%d' % (g, last)].reshape(1, D))

    return _tc_final(
        readouts[0], readouts[1], readouts[2],
        p['lin1_W'], p['lin1_b'].reshape(1, -1),
        p['lin2_W'], p['lin2_b'].reshape(1, -1),
        p['lin3_W'], p['lin3_b'].reshape(1, -1))
